# Initial kernel scaffold; baseline (speedup 1.0000x reference)
import jax
import jax.numpy as jnp
from jax.experimental import pallas as pl


def kernel(x, label_table, Wl, bl, rgb_table, W1, b1, W2, b2, W3, b3, W4, b4):
    raise NotImplementedError("write your pallas kernel here")


# speedup vs baseline: 34.2422x; 34.2422x over previous
import functools

import jax
import jax.numpy as jnp
from jax import lax
from jax.experimental import pallas as pl
from jax.experimental.pallas import tpu as pltpu
from jax.experimental.pallas import tpu_sc as plsc

_N_RAYS = 8192
_NPTS = 200
_TS = 2 ** 19
_MASK = _TS - 1
_PRIME1 = -1640531535
_PRIME2 = 805459861
_NTILES = 32
_RPT = _N_RAYS // _NTILES
_NG = _RPT // 16

def _t_vals():
    return jnp.linspace(0.0, 1.0, _NPTS)


def _grid_arrays(xn, yn, zn, level):
    r = float(16 * (2 ** level) - 1)
    px, py, pz = xn * r, yn * r, zn * r
    ix = px.astype(jnp.int32)
    iy = py.astype(jnp.int32)
    iz = pz.astype(jnp.int32)
    fx = px - ix.astype(jnp.float32)
    fy = py - iy.astype(jnp.float32)
    fz = pz - iz.astype(jnp.float32)
    return ix, iy, iz, fx, fy, fz


def _corner_hashes_weights(ix, iy, iz, fx, fy, fz, level_base):
    p1 = jnp.int32(_PRIME1)
    p2 = jnp.int32(_PRIME2)
    hx = (ix, ix + 1)
    hy0 = iy * p1
    hy = (hy0, hy0 + p1)
    hz0 = iz * p2
    hz = (hz0, hz0 + p2)
    wx = (1.0 - fx, fx)
    wy = (1.0 - fy, fy)
    wz = (1.0 - fz, fz)
    hashes, weights = [], []
    for c in range(8):
        bx, by, bz = c & 1, (c >> 1) & 1, (c >> 2) & 1
        h = ((hx[bx] ^ hy[by] ^ hz[bz]) & jnp.int32(_MASK)) + jnp.int32(level_base)
        w = (wx[bx] * wy[by]) * wz[bz]
        hashes.append(h)
        weights.append(w)
    return hashes, weights


def _make_sc_kernel():
    mesh = plsc.VectorSubcoreMesh(core_axis_name="c", subcore_axis_name="s")

    @functools.partial(
        pl.kernel,
        mesh=mesh,
        out_type=[
            jax.ShapeDtypeStruct((512, 6, _NPTS, 16), jnp.float32),
        ],
        scratch_types=[
            pltpu.VMEM((_NPTS, 3, 16), jnp.float32),
            pltpu.VMEM((128,), jnp.int32),
            pltpu.VMEM((128,), jnp.float32),
            pltpu.VMEM((_NPTS, 16), jnp.float32),
            pltpu.SemaphoreType.DMA,
        ],
    )
    def sc_kernel(pointsg, ltab,
                  out_lg, pbuf, idxb, valb, sbuf, sem):
        cid = lax.axis_index("c")
        sid = lax.axis_index("s")
        wid = cid * 16 + sid

        def group_body(g, carry_g):
            gid = wid * _NG + g
            pltpu.sync_copy(pointsg.at[gid], pbuf)
            for l in range(6):
                def point_body(p, carry, l=l):
                    x = pbuf[p, 0, ...]
                    y = pbuf[p, 1, ...]
                    z = pbuf[p, 2, ...]
                    ix, iy, iz, fx, fy, fz = _grid_arrays(x, y, z, l)
                    hs, ws = _corner_hashes_weights(ix, iy, iz, fx, fy, fz,
                                                    l * _TS)
                    for c in range(8):
                        idxb[pl.ds(c * 16, 16)] = hs[c]
                    pltpu.async_copy(ltab.at[idxb], valb, sem).wait()
                    feat = None
                    for c in range(8):
                        term = ws[c] * valb[pl.ds(c * 16, 16)]
                        feat = term if feat is None else feat + term
                    sbuf[p, ...] = feat
                    return carry

                lax.fori_loop(0, _NPTS, point_body, 0)
                pltpu.sync_copy(sbuf, out_lg.at[gid, l])
            return carry_g

        lax.fori_loop(0, _NG, group_body, 0)

    return sc_kernel


def _make_rgb_kernel():
    mesh = plsc.VectorSubcoreMesh(core_axis_name="c", subcore_axis_name="s")

    @functools.partial(
        pl.kernel,
        mesh=mesh,
        out_type=[
            jax.ShapeDtypeStruct((512, 24, 16), jnp.float32),
        ],
        scratch_types=[
            pltpu.VMEM((3, 16), jnp.float32),
            pltpu.VMEM((128,), jnp.int32),
            pltpu.VMEM((128,), jnp.float32),
            pltpu.VMEM((24, 16), jnp.float32),
            pltpu.SemaphoreType.DMA,
        ],
    )
    def rgb_kernel(selg, rtab, out_enc, pbuf, idxb, valb, ebuf, sem):
        cid = lax.axis_index("c")
        sid = lax.axis_index("s")
        wid = cid * 16 + sid

        def group_body(g, carry_g):
            gid = wid * _NG + g
            pltpu.sync_copy(selg.at[gid], pbuf)
            sx = pbuf[0, ...]
            sy = pbuf[1, ...]
            sz = pbuf[2, ...]
            for l in range(6):
                ix, iy, iz, fx, fy, fz = _grid_arrays(sx, sy, sz, l)
                hs, ws = _corner_hashes_weights(ix, iy, iz, fx, fy, fz,
                                                l * _TS)
                h4 = [h * 4 for h in hs]
                for d in range(4):
                    for c in range(8):
                        idxb[pl.ds(c * 16, 16)] = h4[c] + d
                    pltpu.async_copy(rtab.at[idxb], valb, sem).wait()
                    feat = None
                    for c in range(8):
                        term = ws[c] * valb[pl.ds(c * 16, 16)]
                        feat = term if feat is None else feat + term
                    ebuf[l * 4 + d, ...] = feat
            pltpu.sync_copy(ebuf, out_enc.at[gid])
            return carry_g

        lax.fori_loop(0, _NG, group_body, 0)

    return rgb_kernel


def _mlp_body(xr, w1, b1, w2, b2, w3, b3, w4, b4, outr):
    hp = jax.lax.Precision.HIGHEST
    h = xr[...]
    h = jnp.maximum(jnp.dot(h, w1[...], precision=hp) + b1[...], 0.0)
    h = jnp.maximum(jnp.dot(h, w2[...], precision=hp) + b2[...], 0.0)
    h = jnp.maximum(jnp.dot(h, w3[...], precision=hp) + b3[...], 0.0)
    outr[...] = jnp.dot(h, w4[...], precision=hp) + b4[...]


def _mlp(feat128, w1, b1, w2, b2, w3, b3, w4, b4):
    blk = 1024
    grid = _N_RAYS // blk
    wspec = pl.BlockSpec((128, 128), lambda i: (0, 0))
    bspec = pl.BlockSpec((1, 128), lambda i: (0, 0))
    return pl.pallas_call(
        _mlp_body,
        grid=(grid,),
        in_specs=[pl.BlockSpec((blk, 128), lambda i: (i, 0)),
                  wspec, bspec, wspec, bspec, wspec, bspec, wspec, bspec],
        out_specs=pl.BlockSpec((blk, 128), lambda i: (i, 0)),
        out_shape=jax.ShapeDtypeStruct((_N_RAYS, 128), jnp.float32),
    )(feat128, w1, b1, w2, b2, w3, b3, w4, b4)


def _pad2(w, rows, cols):
    return jnp.pad(w, ((0, rows - w.shape[0]), (0, cols - w.shape[1])))


def kernel(x, label_table, Wl, bl, rgb_table, W1, b1, W2, b2, W3, b3, W4, b4):
    st1 = jnp.sin(x[:, 0])
    p1 = jnp.stack([st1 * jnp.cos(x[:, 1]), st1 * jnp.sin(x[:, 1]),
                    jnp.cos(x[:, 0])], axis=-1)
    st2 = jnp.sin(x[:, 2])
    p2 = jnp.stack([st2 * jnp.cos(x[:, 3]), st2 * jnp.sin(x[:, 3]),
                    jnp.cos(x[:, 2])], axis=-1)
    diff = p2 - p1
    length = jnp.clip(jnp.linalg.norm(diff, axis=-1, keepdims=True), 1e-6, 1e6)
    dirs = diff / length

    ltab = label_table.ravel()
    rtab = rgb_table.ravel()

    points = p1[:, None, :] + diff[:, None, :] * _t_vals().reshape(1, _NPTS, 1)
    xnp = jnp.clip((points + 1.0) * 0.5, 0.0, 1.0)
    xng = xnp.reshape(512, 16, _NPTS, 3).transpose(0, 2, 3, 1)
    (lg,) = _make_sc_kernel()(xng, ltab)
    enc_l = lg.transpose(0, 3, 2, 1).reshape(_N_RAYS, _NPTS, 6)

    output = jax.nn.sigmoid(enc_l @ Wl + bl)
    output_hits = jnp.max(output, axis=1)
    mask = (output > 0.5).astype(jnp.float32)
    idxf = jnp.argmax(mask, axis=1).reshape(-1)
    sel = points[jnp.arange(_N_RAYS), idxf]
    xns = jnp.clip((sel + 1.0) * 0.5, 0.0, 1.0)
    selg = xns.reshape(512, 16, 3).transpose(0, 2, 1)
    (encr,) = _make_rgb_kernel()(selg, rtab)

    feat = jnp.concatenate(
        [dirs, encr.transpose(0, 2, 1).reshape(_N_RAYS, 24)], axis=-1)
    feat128 = jnp.pad(feat, ((0, 0), (0, 101)))
    w1p = _pad2(W1, 128, 128)
    w2p = _pad2(W2, 128, 128)
    w3p = _pad2(W3, 128, 128)
    w4p = _pad2(W4, 128, 128)
    b1p = jnp.pad(b1, (0, 64)).reshape(1, 128)
    b2p = jnp.pad(b2, (0, 64)).reshape(1, 128)
    b3p = jnp.pad(b3, (0, 64)).reshape(1, 128)
    b4p = jnp.pad(b4, (0, 125)).reshape(1, 128)
    rgb128 = _mlp(feat128, w1p, b1p, w2p, b2p, w3p, b3p, w4p, b4p)
    output_rgb = rgb128[:, :3]
    return (output_hits, output, idxf, output_rgb)
